# baseline (speedup 1.0000x reference)
import jax
import jax.numpy as jnp
from jax.experimental import pallas as pl
from jax.experimental.pallas import tpu as pltpu
from jax.experimental.pallas import tpu_sc as plsc

B = 4096
L = 50
D = 128
G = 4 * D
TIME_NUM = 50
TIME_PAD = 64
MAX_TIME = 1000.0
GATHER_WINDOW = 128
PAYLOAD = 128
NC = 8
BC = B // NC
HB = BC // 2

_MESH = plsc.VectorSubcoreMesh(core_axis_name="core", subcore_axis_name="subcore")


def _sc_gather(table, flat_idx):
    n = flat_idx.shape[0]
    idx2d = flat_idx.reshape(1, n)

    @pl.kernel(
        out_type=jax.ShapeDtypeStruct((n, table.shape[1]), table.dtype),
        mesh=_MESH,
    )
    def kern(x_hbm, i_hbm, o_hbm):
        def body(i_vmem, o_vmem):
            pltpu.sync_copy(x_hbm.at[i_vmem.at[0]], o_vmem)

        pltpu.emit_pipeline(
            body,
            grid=(n // GATHER_WINDOW,),
            in_specs=[pl.BlockSpec((1, GATHER_WINDOW), index_map=lambda i: (0, i))],
            out_specs=[
                pl.BlockSpec((GATHER_WINDOW, table.shape[1]), index_map=lambda i: (i, 0))
            ],
            core_axis_name=("core", "subcore"),
            dimension_semantics=(pltpu.PARALLEL,),
        )(i_hbm, o_hbm)

    return kern(table, idx2d)


def _sc_scatter(data, flat_idx):
    n = flat_idx.shape[0]
    idx2d = flat_idx.reshape(1, n)

    @pl.kernel(
        out_type=jax.ShapeDtypeStruct(data.shape, data.dtype),
        mesh=_MESH,
    )
    def kern(x_hbm, i_hbm, o_hbm):
        def body(x_vmem, i_vmem):
            pltpu.sync_copy(x_vmem, o_hbm.at[i_vmem.at[0]])

        pltpu.emit_pipeline(
            body,
            grid=(n // GATHER_WINDOW,),
            in_specs=[
                pl.BlockSpec((GATHER_WINDOW, data.shape[1]), index_map=lambda i: (i, 0)),
                pl.BlockSpec((1, GATHER_WINDOW), index_map=lambda i: (0, i)),
            ],
            out_specs=[],
            core_axis_name=("core", "subcore"),
            dimension_semantics=(pltpu.PARALLEL,),
        )(x_hbm, i_hbm)

    return kern(data, idx2d)


def _sigmoid(x):
    return 0.5 * jnp.tanh(0.5 * x) + 0.5


def _lstm_kernel(maxlen_ref, x_ref, tidx_ref, len_ref, biasg_ref, timeg_ref,
                 wih_ref, whh_ref, wtr_ref, btr_ref, out_ref):
    c_id = pl.program_id(0)
    m = maxlen_ref[c_id]
    lenv = len_ref[...]
    iota = jax.lax.broadcasted_iota(jnp.int32, (BC, TIME_PAD), 1)

    def step(t, carry):
        h, c = carry
        xt = x_ref[t].astype(jnp.bfloat16)
        tcol = tidx_ref[t]
        onehot = (tcol == iota).astype(jnp.bfloat16)
        gates = (jnp.dot(xt, wih_ref[...],
                         preferred_element_type=jnp.float32)
                 + jnp.dot(h.astype(jnp.bfloat16), whh_ref[...],
                           preferred_element_type=jnp.float32)
                 + jnp.dot(onehot, timeg_ref[...],
                           preferred_element_type=jnp.float32)
                 + biasg_ref[t])
        gi = _sigmoid(gates[:, 0:D])
        gf = _sigmoid(gates[:, D:2 * D])
        gg = jnp.tanh(gates[:, 2 * D:3 * D])
        go = _sigmoid(gates[:, 3 * D:4 * D])
        c_new = gf * c + gi * gg
        h_new = go * jnp.tanh(c_new)
        mask = t < lenv
        return jnp.where(mask, h_new, h), jnp.where(mask, c_new, c)

    h0 = jnp.zeros((BC, D), jnp.float32)
    h, _ = jax.lax.fori_loop(0, m, step, (h0, h0))
    out_ref[...] = jax.nn.relu(
        jnp.dot(h.astype(jnp.bfloat16), wtr_ref[...],
                preferred_element_type=jnp.float32)
        + btr_ref[...])


def _run_lstm(maxlen, x_lbd, tidx_t, len2d, biasg, timeg, wih_t, whh_t,
              wtr_t, btr):
    nch = maxlen.shape[0]
    nb = nch * BC
    grid_spec = pltpu.PrefetchScalarGridSpec(
        num_scalar_prefetch=1,
        grid=(nch,),
        in_specs=[
            pl.BlockSpec((L, BC, D), lambda c, m: (0, c, 0)),
            pl.BlockSpec((L, BC, 1), lambda c, m: (0, c, 0)),
            pl.BlockSpec((BC, 1), lambda c, m: (c, 0)),
            pl.BlockSpec((L, 1, G), lambda c, m: (0, 0, 0)),
            pl.BlockSpec((TIME_PAD, G), lambda c, m: (0, 0)),
            pl.BlockSpec((D, G), lambda c, m: (0, 0)),
            pl.BlockSpec((D, G), lambda c, m: (0, 0)),
            pl.BlockSpec((D, D), lambda c, m: (0, 0)),
            pl.BlockSpec((1, D), lambda c, m: (0, 0)),
        ],
        out_specs=pl.BlockSpec((BC, D), lambda c, m: (c, 0)),
        scratch_shapes=[],
    )
    return pl.pallas_call(
        _lstm_kernel,
        grid_spec=grid_spec,
        out_shape=jax.ShapeDtypeStruct((nb, D), jnp.float32),
        compiler_params=pltpu.CompilerParams(
            dimension_semantics=("arbitrary",)),
    )(maxlen, x_lbd, tidx_t, len2d, biasg, timeg, wih_t, whh_t, wtr_t, btr)


def kernel(static_table, time_table, pos_table, W_ih, W_hh, b_ih, b_hh,
           W_trans, b_trans, cas_times, cas_history, lengths):
    vals = jax.lax.broadcasted_iota(jnp.int32, (B, TIME_PAD), 1)
    onehot = (lengths[:, None] == vals).astype(jnp.int32)
    cum = jnp.cumsum(onehot, axis=0)
    hist = cum[-1]
    offset = jnp.cumsum(hist) - hist
    inccum = offset + hist
    pos = jnp.sum(onehot * (offset[None, :] + cum), axis=1) - 1
    thresholds = BC * (jnp.arange(NC, dtype=jnp.int32) + 1)
    maxlen = jnp.sum((inccum[None, :] < thresholds[:, None]).astype(jnp.int32),
                     axis=1)

    tidx = jnp.clip(
        jnp.floor(cas_times / MAX_TIME * TIME_NUM).astype(jnp.int32),
        0, TIME_NUM - 1)
    payload = jnp.concatenate(
        [cas_history, tidx,
         lengths.reshape(B, 1),
         jnp.zeros((B, PAYLOAD - 2 * L - 1), jnp.int32)], axis=1)
    payload_s = _sc_scatter(payload, pos)

    both_t = payload_s[:, :2 * L].T

    wih_f = W_ih.T.astype(jnp.float32)
    timeg = jnp.zeros((TIME_PAD, G), jnp.float32).at[:TIME_NUM].set(
        time_table @ wih_f).astype(jnp.bfloat16)
    biasg = (pos_table[:L] @ wih_f + b_ih + b_hh).reshape(L, 1, G)
    wih_b = W_ih.T.astype(jnp.bfloat16)
    whh_b = W_hh.T.astype(jnp.bfloat16)
    wtr_b = W_trans.T.astype(jnp.bfloat16)
    btr = b_trans.reshape(1, D)

    nsp = 2
    bh = B // nsp
    nch = NC // nsp
    xs = []
    for s in range(nsp):
        idx_s = both_t[:L, s * bh:(s + 1) * bh].reshape(L * bh)
        xs.append(_sc_gather(static_table, idx_s).reshape(L, bh, D))
    outs = []
    for s in range(nsp):
        sl = slice(s * bh, (s + 1) * bh)
        outs.append(_run_lstm(
            maxlen[s * nch:(s + 1) * nch], xs[s],
            both_t[L:2 * L, sl].reshape(L, bh, 1),
            payload_s[sl, 2 * L:2 * L + 1],
            biasg, timeg, wih_b, whh_b, wtr_b, btr))
    out_s = jnp.concatenate(outs, axis=0)
    return _sc_gather(out_s, pos)


# speedup vs baseline: 1.0348x; 1.0348x over previous
import jax
import jax.numpy as jnp
from jax.experimental import pallas as pl
from jax.experimental.pallas import tpu as pltpu
from jax.experimental.pallas import tpu_sc as plsc

B = 4096
L = 50
D = 128
G = 4 * D
TIME_NUM = 50
TIME_PAD = 64
MAX_TIME = 1000.0
GATHER_WINDOW = 128

_MESH = plsc.VectorSubcoreMesh(core_axis_name="core", subcore_axis_name="subcore")


def _sc_gather(table, flat_idx):
    n = flat_idx.shape[0]
    idx2d = flat_idx.reshape(1, n)

    @pl.kernel(
        out_type=jax.ShapeDtypeStruct((n, table.shape[1]), table.dtype),
        mesh=_MESH,
    )
    def kern(x_hbm, i_hbm, o_hbm):
        def body(i_vmem, o_vmem):
            pltpu.sync_copy(x_hbm.at[i_vmem.at[0]], o_vmem)

        pltpu.emit_pipeline(
            body,
            grid=(n // GATHER_WINDOW,),
            in_specs=[pl.BlockSpec((1, GATHER_WINDOW), index_map=lambda i: (0, i))],
            out_specs=[
                pl.BlockSpec((GATHER_WINDOW, table.shape[1]), index_map=lambda i: (i, 0))
            ],
            core_axis_name=("core", "subcore"),
            dimension_semantics=(pltpu.PARALLEL,),
        )(i_hbm, o_hbm)

    return kern(table, idx2d)


def _sigmoid(x):
    return 0.5 * jnp.tanh(0.5 * x) + 0.5


def _lstm_kernel(x_ref, tidx_ref, len_ref, biasg_ref, timeg_ref,
                 wih_ref, whh_ref, wtr_ref, btr_ref, out_ref, h_ref, c_ref):
    t = pl.program_id(0)

    @pl.when(t == 0)
    def _():
        h_ref[...] = jnp.zeros_like(h_ref)
        c_ref[...] = jnp.zeros_like(c_ref)

    xt = x_ref[0].astype(jnp.bfloat16)
    tcol = tidx_ref[0]
    onehot = (tcol == jax.lax.broadcasted_iota(
        jnp.int32, (B, TIME_PAD), 1)).astype(jnp.bfloat16)
    h = h_ref[...]
    c = c_ref[...]
    gates = (jnp.dot(xt, wih_ref[...], preferred_element_type=jnp.float32)
             + jnp.dot(h.astype(jnp.bfloat16), whh_ref[...],
                       preferred_element_type=jnp.float32)
             + jnp.dot(onehot, timeg_ref[...],
                       preferred_element_type=jnp.float32)
             + biasg_ref[0])
    gi = _sigmoid(gates[:, 0:D])
    gf = _sigmoid(gates[:, D:2 * D])
    gg = jnp.tanh(gates[:, 2 * D:3 * D])
    go = _sigmoid(gates[:, 3 * D:4 * D])
    c_new = gf * c + gi * gg
    h_new = go * jnp.tanh(c_new)
    mask = t < len_ref[...]
    h = jnp.where(mask, h_new, h)
    h_ref[...] = h
    c_ref[...] = jnp.where(mask, c_new, c)

    @pl.when(t == L - 1)
    def _():
        out_ref[...] = jax.nn.relu(
            jnp.dot(h.astype(jnp.bfloat16), wtr_ref[...],
                    preferred_element_type=jnp.float32)
            + btr_ref[...])


def _run_lstm(x_lbd, tidx_t, len2d, biasg, timeg, wih_t, whh_t, wtr_t, btr):
    return pl.pallas_call(
        _lstm_kernel,
        grid=(L,),
        in_specs=[
            pl.BlockSpec((1, B, D), lambda t: (t, 0, 0)),
            pl.BlockSpec((1, B, 1), lambda t: (t, 0, 0)),
            pl.BlockSpec((B, 1), lambda t: (0, 0)),
            pl.BlockSpec((1, 1, G), lambda t: (t, 0, 0)),
            pl.BlockSpec((TIME_PAD, G), lambda t: (0, 0)),
            pl.BlockSpec((D, G), lambda t: (0, 0)),
            pl.BlockSpec((D, G), lambda t: (0, 0)),
            pl.BlockSpec((D, D), lambda t: (0, 0)),
            pl.BlockSpec((1, D), lambda t: (0, 0)),
        ],
        out_specs=pl.BlockSpec((B, D), lambda t: (0, 0)),
        out_shape=jax.ShapeDtypeStruct((B, D), jnp.float32),
        scratch_shapes=[
            pltpu.VMEM((B, D), jnp.float32),
            pltpu.VMEM((B, D), jnp.float32),
        ],
        compiler_params=pltpu.CompilerParams(
            dimension_semantics=("arbitrary",)),
    )(x_lbd, tidx_t, len2d, biasg, timeg, wih_t, whh_t, wtr_t, btr)


def kernel(static_table, time_table, pos_table, W_ih, W_hh, b_ih, b_hh,
           W_trans, b_trans, cas_times, cas_history, lengths):
    tidx = jnp.clip(
        jnp.floor(cas_times / MAX_TIME * TIME_NUM).astype(jnp.int32),
        0, TIME_NUM - 1)
    tidx_t = tidx.T.reshape(L, B, 1)
    idx_flat = cas_history.T.reshape(L * B)
    x_lbd = _sc_gather(static_table, idx_flat).reshape(L, B, D)

    wih_f = W_ih.T.astype(jnp.float32)
    timeg = jnp.zeros((TIME_PAD, G), jnp.float32).at[:TIME_NUM].set(
        time_table @ wih_f).astype(jnp.bfloat16)
    biasg = (pos_table[:L] @ wih_f + b_ih + b_hh).reshape(L, 1, G)

    return _run_lstm(x_lbd, tidx_t, lengths.reshape(B, 1), biasg, timeg,
                     W_ih.T.astype(jnp.bfloat16), W_hh.T.astype(jnp.bfloat16),
                     W_trans.T.astype(jnp.bfloat16), b_trans.reshape(1, D))


# speedup vs baseline: 1.1262x; 1.0883x over previous
import jax
import jax.numpy as jnp
from jax.experimental import pallas as pl
from jax.experimental.pallas import tpu as pltpu
from jax.experimental.pallas import tpu_sc as plsc

B = 4096
L = 50
D = 128
G = 4 * D
TIME_NUM = 50
TIME_PAD = 64
MAX_TIME = 1000.0
GATHER_WINDOW = 128

_MESH = plsc.VectorSubcoreMesh(core_axis_name="core", subcore_axis_name="subcore")


def _sc_gather(table, flat_idx):
    n = flat_idx.shape[0]
    idx2d = flat_idx.reshape(1, n)

    @pl.kernel(
        out_type=jax.ShapeDtypeStruct((n, table.shape[1]), table.dtype),
        mesh=_MESH,
    )
    def kern(x_hbm, i_hbm, o_hbm):
        def body(i_vmem, o_vmem):
            pltpu.sync_copy(x_hbm.at[i_vmem.at[0]], o_vmem)

        pltpu.emit_pipeline(
            body,
            grid=(n // GATHER_WINDOW,),
            in_specs=[pl.BlockSpec((1, GATHER_WINDOW), index_map=lambda i: (0, i))],
            out_specs=[
                pl.BlockSpec((GATHER_WINDOW, table.shape[1]), index_map=lambda i: (i, 0))
            ],
            core_axis_name=("core", "subcore"),
            dimension_semantics=(pltpu.PARALLEL,),
        )(i_hbm, o_hbm)

    return kern(table, idx2d)


def _sigmoid(x):
    return 0.5 * jnp.tanh(0.5 * x) + 0.5


def _lstm_kernel(x_ref, tidx_ref, len_ref, biasg_ref, timeg_ref,
                 wih_ref, whh_ref, wtr_ref, btr_ref, out_ref, h_ref, c_ref):
    t = pl.program_id(0)

    @pl.when(t == 0)
    def _():
        h_ref[...] = jnp.zeros_like(h_ref)
        c_ref[...] = jnp.zeros_like(c_ref)

    tcol = tidx_ref[0]
    onehot = (tcol == jax.lax.broadcasted_iota(
        jnp.int32, (B, TIME_PAD), 1)).astype(jnp.bfloat16)
    xt = (x_ref[0] + jnp.dot(onehot, timeg_ref[...],
                             preferred_element_type=jnp.float32)
          ).astype(jnp.bfloat16)
    h = h_ref[...]
    c = c_ref[...]
    gates = (jnp.dot(xt, wih_ref[...], preferred_element_type=jnp.float32)
             + jnp.dot(h.astype(jnp.bfloat16), whh_ref[...],
                       preferred_element_type=jnp.float32)
             + biasg_ref[0])
    gi = _sigmoid(gates[:, 0:D])
    gf = _sigmoid(gates[:, D:2 * D])
    gg = jnp.tanh(gates[:, 2 * D:3 * D])
    go = _sigmoid(gates[:, 3 * D:4 * D])
    c_new = gf * c + gi * gg
    h_new = go * jnp.tanh(c_new)
    mask = t < len_ref[...]
    h = jnp.where(mask, h_new, h)
    h_ref[...] = h
    c_ref[...] = jnp.where(mask, c_new, c)

    @pl.when(t == L - 1)
    def _():
        out_ref[...] = jax.nn.relu(
            jnp.dot(h.astype(jnp.bfloat16), wtr_ref[...],
                    preferred_element_type=jnp.float32)
            + btr_ref[...])


def _run_lstm(x_lbd, tidx_t, len2d, biasg, timeg, wih_t, whh_t, wtr_t, btr):
    return pl.pallas_call(
        _lstm_kernel,
        grid=(L,),
        in_specs=[
            pl.BlockSpec((1, B, D), lambda t: (t, 0, 0)),
            pl.BlockSpec((1, B, 1), lambda t: (t, 0, 0)),
            pl.BlockSpec((B, 1), lambda t: (0, 0)),
            pl.BlockSpec((1, 1, G), lambda t: (t, 0, 0)),
            pl.BlockSpec((TIME_PAD, D), lambda t: (0, 0)),
            pl.BlockSpec((D, G), lambda t: (0, 0)),
            pl.BlockSpec((D, G), lambda t: (0, 0)),
            pl.BlockSpec((D, D), lambda t: (0, 0)),
            pl.BlockSpec((1, D), lambda t: (0, 0)),
        ],
        out_specs=pl.BlockSpec((B, D), lambda t: (0, 0)),
        out_shape=jax.ShapeDtypeStruct((B, D), jnp.float32),
        scratch_shapes=[
            pltpu.VMEM((B, D), jnp.float32),
            pltpu.VMEM((B, D), jnp.float32),
        ],
        compiler_params=pltpu.CompilerParams(
            dimension_semantics=("arbitrary",)),
    )(x_lbd, tidx_t, len2d, biasg, timeg, wih_t, whh_t, wtr_t, btr)


def kernel(static_table, time_table, pos_table, W_ih, W_hh, b_ih, b_hh,
           W_trans, b_trans, cas_times, cas_history, lengths):
    tidx = jnp.clip(
        jnp.floor(cas_times / MAX_TIME * TIME_NUM).astype(jnp.int32),
        0, TIME_NUM - 1)
    tidx_t = tidx.T.reshape(L, B, 1)
    idx_flat = cas_history.T.reshape(L * B)
    x_lbd = _sc_gather(static_table, idx_flat).reshape(L, B, D)

    wih_f = W_ih.T.astype(jnp.float32)
    timeg = jnp.zeros((TIME_PAD, D), jnp.float32).at[:TIME_NUM].set(
        time_table).astype(jnp.bfloat16)
    biasg = (pos_table[:L] @ wih_f + b_ih + b_hh).reshape(L, 1, G)

    return _run_lstm(x_lbd, tidx_t, lengths.reshape(B, 1), biasg, timeg,
                     W_ih.T.astype(jnp.bfloat16), W_hh.T.astype(jnp.bfloat16),
                     W_trans.T.astype(jnp.bfloat16), b_trans.reshape(1, D))


# speedup vs baseline: 1.1294x; 1.0028x over previous
import jax
import jax.numpy as jnp
from jax.experimental import pallas as pl
from jax.experimental.pallas import tpu as pltpu
from jax.experimental.pallas import tpu_sc as plsc

B = 4096
L = 50
D = 128
G = 4 * D
TIME_NUM = 50
TIME_PAD = 64
MAX_TIME = 1000.0
GATHER_WINDOW = 256

_MESH = plsc.VectorSubcoreMesh(core_axis_name="core", subcore_axis_name="subcore")


def _sc_gather(table, flat_idx):
    n = flat_idx.shape[0]
    idx2d = flat_idx.reshape(1, n)

    @pl.kernel(
        out_type=jax.ShapeDtypeStruct((n, table.shape[1]), table.dtype),
        mesh=_MESH,
    )
    def kern(x_hbm, i_hbm, o_hbm):
        def body(i_vmem, o_vmem):
            pltpu.sync_copy(x_hbm.at[i_vmem.at[0]], o_vmem)

        pltpu.emit_pipeline(
            body,
            grid=(n // GATHER_WINDOW,),
            in_specs=[pl.BlockSpec((1, GATHER_WINDOW), index_map=lambda i: (0, i))],
            out_specs=[
                pl.BlockSpec((GATHER_WINDOW, table.shape[1]), index_map=lambda i: (i, 0))
            ],
            core_axis_name=("core", "subcore"),
            dimension_semantics=(pltpu.PARALLEL,),
        )(i_hbm, o_hbm)

    return kern(table, idx2d)


def _sigmoid(x):
    return 0.5 * jnp.tanh(0.5 * x) + 0.5


def _lstm_kernel(x_ref, tidx_ref, len_ref, biasg_ref, timeg_ref,
                 wih_ref, whh_ref, wtr_ref, btr_ref, out_ref, h_ref, c_ref):
    t = pl.program_id(0)

    @pl.when(t == 0)
    def _():
        h_ref[...] = jnp.zeros_like(h_ref)
        c_ref[...] = jnp.zeros_like(c_ref)

    tcol = tidx_ref[0]
    onehot = (tcol == jax.lax.broadcasted_iota(
        jnp.int32, (B, TIME_PAD), 1)).astype(jnp.bfloat16)
    xt = (x_ref[0] + jnp.dot(onehot, timeg_ref[...],
                             preferred_element_type=jnp.float32)
          ).astype(jnp.bfloat16)
    h = h_ref[...]
    c = c_ref[...]
    gates = (jnp.dot(xt, wih_ref[...], preferred_element_type=jnp.float32)
             + jnp.dot(h.astype(jnp.bfloat16), whh_ref[...],
                       preferred_element_type=jnp.float32)
             + biasg_ref[0])
    gi = _sigmoid(gates[:, 0:D])
    gf = _sigmoid(gates[:, D:2 * D])
    gg = jnp.tanh(gates[:, 2 * D:3 * D])
    go = _sigmoid(gates[:, 3 * D:4 * D])
    c_new = gf * c + gi * gg
    h_new = go * jnp.tanh(c_new)
    mask = t < len_ref[...]
    h = jnp.where(mask, h_new, h)
    h_ref[...] = h
    c_ref[...] = jnp.where(mask, c_new, c)

    @pl.when(t == L - 1)
    def _():
        out_ref[...] = jax.nn.relu(
            jnp.dot(h.astype(jnp.bfloat16), wtr_ref[...],
                    preferred_element_type=jnp.float32)
            + btr_ref[...])


def _run_lstm(x_lbd, tidx_t, len2d, biasg, timeg, wih_t, whh_t, wtr_t, btr):
    return pl.pallas_call(
        _lstm_kernel,
        grid=(L,),
        in_specs=[
            pl.BlockSpec((1, B, D), lambda t: (t, 0, 0)),
            pl.BlockSpec((1, B, 1), lambda t: (t, 0, 0)),
            pl.BlockSpec((B, 1), lambda t: (0, 0)),
            pl.BlockSpec((1, 1, G), lambda t: (t, 0, 0)),
            pl.BlockSpec((TIME_PAD, D), lambda t: (0, 0)),
            pl.BlockSpec((D, G), lambda t: (0, 0)),
            pl.BlockSpec((D, G), lambda t: (0, 0)),
            pl.BlockSpec((D, D), lambda t: (0, 0)),
            pl.BlockSpec((1, D), lambda t: (0, 0)),
        ],
        out_specs=pl.BlockSpec((B, D), lambda t: (0, 0)),
        out_shape=jax.ShapeDtypeStruct((B, D), jnp.float32),
        scratch_shapes=[
            pltpu.VMEM((B, D), jnp.float32),
            pltpu.VMEM((B, D), jnp.float32),
        ],
        compiler_params=pltpu.CompilerParams(
            dimension_semantics=("arbitrary",)),
    )(x_lbd, tidx_t, len2d, biasg, timeg, wih_t, whh_t, wtr_t, btr)


def kernel(static_table, time_table, pos_table, W_ih, W_hh, b_ih, b_hh,
           W_trans, b_trans, cas_times, cas_history, lengths):
    tidx = jnp.clip(
        jnp.floor(cas_times / MAX_TIME * TIME_NUM).astype(jnp.int32),
        0, TIME_NUM - 1)
    tidx_t = tidx.T.reshape(L, B, 1)
    idx_flat = cas_history.T.reshape(L * B)
    x_lbd = _sc_gather(static_table, idx_flat).reshape(L, B, D)

    wih_f = W_ih.T.astype(jnp.float32)
    timeg = jnp.zeros((TIME_PAD, D), jnp.float32).at[:TIME_NUM].set(
        time_table).astype(jnp.bfloat16)
    biasg = (pos_table[:L] @ wih_f + b_ih + b_hh).reshape(L, 1, G)

    return _run_lstm(x_lbd, tidx_t, lengths.reshape(B, 1), biasg, timeg,
                     W_ih.T.astype(jnp.bfloat16), W_hh.T.astype(jnp.bfloat16),
                     W_trans.T.astype(jnp.bfloat16), b_trans.reshape(1, D))
